# Initial kernel scaffold; baseline (speedup 1.0000x reference)
import jax
import jax.numpy as jnp
from jax.experimental import pallas as pl


def kernel(x_bus, x_gmd_bus, edge_index_bus_conn_bus, edge_index_bus_to_gmd_bus, edge_index_gmd_bus_from_bus, params):
    raise NotImplementedError("write your pallas kernel here")


# speedup vs baseline: 17.9905x; 17.9905x over previous
import functools

import jax
import jax.numpy as jnp
import numpy as np
from jax import lax
from jax.experimental import pallas as pl
from jax.experimental.pallas import tpu as pltpu
from jax.experimental.pallas import tpu_sc as plsc

N_BUS = 50000
N_GMD = 10000
D_IN = 128
HID = 64
HEADS = 2
DH = HID // HEADS
NC = 2
NS = 16
CHUNK = 128
DEN_W = 8
PREC = lax.Precision.HIGHEST
BN = 1000


def _relu_linear_body(x_ref, w_ref, b_ref, o_ref):
  o_ref[...] = jax.nn.relu(
      jnp.dot(x_ref[...], w_ref[...], precision=PREC) + b_ref[...])


def _embed(x, w, b):
  n, d = x.shape
  return pl.pallas_call(
      _relu_linear_body,
      grid=(n // BN,),
      in_specs=[
          pl.BlockSpec((BN, d), lambda i: (i, 0)),
          pl.BlockSpec((d, HID), lambda i: (0, 0)),
          pl.BlockSpec((1, HID), lambda i: (0, 0)),
      ],
      out_specs=pl.BlockSpec((BN, HID), lambda i: (i, 0)),
      out_shape=jax.ShapeDtypeStruct((n, HID), jnp.float32),
  )(x, w, b.reshape(1, HID))


def _tables_body(nroles, *refs):
  h_ref = refs[0]
  w_refs = refs[1:1 + nroles]
  b_refs = refs[1 + nroles:1 + 2 * nroles]
  o_refs = refs[1 + 2 * nroles:]
  hb = h_ref[...]
  for wr, br, orf in zip(w_refs, b_refs, o_refs):
    orf[...] = (jnp.dot(hb, wr[0], precision=PREC) + br[0])[None]


def _tables(h, ws, bs):
  n = h.shape[0]
  nroles = len(ws)
  grid = (HEADS, n // BN)
  in_specs = [pl.BlockSpec((BN, HID), lambda hh, i: (i, 0))]
  in_specs += [pl.BlockSpec((1, HID, DH), lambda hh, i: (hh, 0, 0))] * nroles
  in_specs += [pl.BlockSpec((1, 1, DH), lambda hh, i: (hh, 0, 0))] * nroles
  out_specs = [pl.BlockSpec((1, BN, DH), lambda hh, i: (hh, i, 0))] * nroles
  out_shape = [jax.ShapeDtypeStruct((HEADS, n, DH), jnp.float32)] * nroles
  outs = pl.pallas_call(
      functools.partial(_tables_body, nroles),
      grid=grid,
      in_specs=in_specs,
      out_specs=out_specs,
      out_shape=out_shape,
  )(h, *ws, *[b.reshape(HEADS, 1, DH) for b in bs])
  return [o.reshape(HEADS * n, DH) for o in outs]


def _combine_body(ne, mlp, *refs):
  num_refs = refs[:ne]
  den_refs = refs[ne:2 * ne]
  ne2 = 2 * ne
  h_ref = refs[ne2]
  wa_ref, ba_ref, skip_ref = refs[ne2 + 1:ne2 + 4]
  o_ref = refs[-1]
  aggs = []
  for nr, dr in zip(num_refs, den_refs):
    num = nr[...]
    den = dr[...][:, :, None]
    agg = num / (den + 1e-16)
    aggs.append(jnp.concatenate([agg[0], agg[1]], axis=-1))
  m = aggs[0]
  for other in aggs[1:]:
    m = jnp.minimum(m, other)
  o = jnp.dot(jax.nn.gelu(m), wa_ref[...], precision=PREC) + ba_ref[...]
  gate = jax.nn.sigmoid(skip_ref[0, 0])
  h2 = gate * o + (1.0 - gate) * h_ref[...]
  if mlp:
    mrefs = refs[ne2 + 4:-1]
    for j in range(0, len(mrefs), 2):
      w, b = mrefs[j], mrefs[j + 1]
      h2 = jnp.dot(h2, w[...], precision=PREC) + b[...]
      if j + 2 < len(mrefs):
        h2 = jax.nn.relu(h2)
  o_ref[...] = h2


def _combine(accs, h_prev, wa, ba, skip, mlp_params=None):
  n = h_prev.shape[0]
  ne = len(accs)
  bc = 1024
  grid = (pl.cdiv(n, bc),)
  in_specs = [pl.BlockSpec((HEADS, bc, DH), lambda i: (0, i, 0))] * ne
  in_specs += [pl.BlockSpec((HEADS, bc), lambda i: (0, i))] * ne
  in_specs += [
      pl.BlockSpec((bc, HID), lambda i: (i, 0)),
      pl.BlockSpec((HID, HID), lambda i: (0, 0)),
      pl.BlockSpec((1, HID), lambda i: (0, 0)),
      pl.BlockSpec(memory_space=pltpu.SMEM),
  ]
  args = [*[a[0] for a in accs], *[a[1] for a in accs],
          h_prev, wa, ba.reshape(1, HID), skip.reshape(1, 1)]
  if mlp_params is not None:
    for w, b in mlp_params:
      dout = w.shape[1]
      in_specs += [
          pl.BlockSpec((HID, dout), lambda i: (0, 0)),
          pl.BlockSpec((1, dout), lambda i: (0, 0)),
      ]
      args += [w, b.reshape(1, dout)]
    dfin = mlp_params[-1][0].shape[1]
  else:
    dfin = HID
  return pl.pallas_call(
      functools.partial(_combine_body, ne, mlp_params is not None),
      grid=grid,
      in_specs=in_specs,
      out_specs=pl.BlockSpec((bc, dfin), lambda i: (i, 0)),
      out_shape=jax.ShapeDtypeStruct((n, dfin), jnp.float32),
  )(*args)


def _edge_sc(src, dst, ktab, vtab, qtab, nsrc, ndst):
  e = src.shape[0]
  estep = NS * CHUNK
  e_pad = ((e + estep - 1) // estep) * estep
  if e_pad != e:
    src = jnp.concatenate([src, jnp.zeros((e_pad - e,), jnp.int32)])
    dst = jnp.concatenate([dst, jnp.full((e_pad - e,), ndst, jnp.int32)])
  ndst_pad = ((ndst + 1 + 127) // 128) * 128
  nden = ndst_pad // DEN_W + 16
  nden = ((nden + 127) // 128) * 128
  r_tile = ndst_pad // NS
  rd_tile = nden // NS
  e_tile = e_pad // NS
  n_chunks = e_tile // CHUNK
  znum = jnp.zeros((r_tile, DH), jnp.float32)
  zden = jnp.zeros((rd_tile, DEN_W), jnp.float32)
  mesh = plsc.VectorSubcoreMesh(
      core_axis_name="c", subcore_axis_name="s",
      num_cores=NC, num_subcores=NS)

  @functools.partial(
      pl.kernel,
      mesh=mesh,
      compiler_params=pltpu.CompilerParams(
          needs_layout_passes=False, use_tc_tiling_on_sc=False),
      out_type=(
          jax.ShapeDtypeStruct((NC, ndst_pad, DH), jnp.float32),
          jax.ShapeDtypeStruct((NC, nden, DEN_W), jnp.float32),
      ),
      scratch_types=[
          pltpu.VMEM((CHUNK,), jnp.int32),
          pltpu.VMEM((CHUNK,), jnp.int32),
          pltpu.VMEM((CHUNK,), jnp.int32),
          pltpu.VMEM((CHUNK,), jnp.int32),
          pltpu.VMEM((CHUNK,), jnp.int32),
          pltpu.VMEM((CHUNK, DH), jnp.float32),
          pltpu.VMEM((CHUNK, DH), jnp.float32),
          pltpu.VMEM((CHUNK, DH), jnp.float32),
          pltpu.VMEM((CHUNK, DEN_W), jnp.float32),
          pltpu.VMEM_SHARED((ndst_pad, DH), jnp.float32),
          pltpu.VMEM_SHARED((nden, DEN_W), jnp.float32),
          pltpu.SemaphoreType.DMA,
          pltpu.SemaphoreType.DMA,
          pltpu.SemaphoreType.DMA,
      ],
  )
  def k(src_hbm, dst_hbm, ktab_hbm, vtab_hbm, qtab_hbm, znum_hbm, zden_hbm,
        onum_hbm, oden_hbm,
        idx_s, idx_d, idx_sg, idx_dg, idx_d8, kr, vr, qr, md, acc_n, acc_d,
        sem0, sem1, sem2):
    c = lax.axis_index("c")
    s = lax.axis_index("s")
    pltpu.sync_copy(znum_hbm, acc_n.at[pl.ds(s * r_tile, r_tile)])
    pltpu.sync_copy(zden_hbm, acc_d.at[pl.ds(s * rd_tile, rd_tile)])
    zv = jnp.zeros((16,), jnp.float32)
    for g in range(CHUNK // 16):
      rows = g * 16 + lax.iota(jnp.int32, 16)
      for ch in range(DEN_W):
        plsc.store_scatter(md, [rows, jnp.full((16,), ch, jnp.int32)], zv)
    plsc.subcore_barrier()
    base_t = s * e_tile

    @pl.loop(0, n_chunks)
    def _chunk(i):
      base = base_t + i * CHUNK
      pltpu.sync_copy(src_hbm.at[pl.ds(base, CHUNK)], idx_s)
      pltpu.sync_copy(dst_hbm.at[pl.ds(base, CHUNK)], idx_d)
      for g in range(CHUNK // 16):
        sl = pl.ds(g * 16, 16)
        idx_sg[sl] = idx_s[sl] + c * nsrc
        idx_dg[sl] = jnp.minimum(idx_d[sl], ndst - 1) + c * ndst
        idx_d8[sl] = idx_d[sl] >> 3
      cp_k = pltpu.async_copy(ktab_hbm.at[idx_sg], kr, sem0)
      cp_v = pltpu.async_copy(vtab_hbm.at[idx_sg], vr, sem1)
      cp_q = pltpu.async_copy(qtab_hbm.at[idx_dg], qr, sem2)
      cp_k.wait()
      cp_q.wait()
      for g in range(CHUNK // 16):
        rows = g * 16 + lax.iota(jnp.int32, 16)
        a = jnp.zeros((16,), jnp.float32)
        for ch in range(DH):
          cc = jnp.full((16,), ch, jnp.int32)
          a = a + (plsc.load_gather(qr, [rows, cc]) *
                   plsc.load_gather(kr, [rows, cc]))
        ev = jnp.exp(a)
        if g == 0:
          cp_v.wait()
        for ch in range(DH):
          cc = jnp.full((16,), ch, jnp.int32)
          plsc.store_scatter(vr, [rows, cc],
                             plsc.load_gather(vr, [rows, cc]) * ev)
        plsc.store_scatter(md, [rows, idx_d[pl.ds(g * 16, 16)] & 7], ev)
      pltpu.sync_copy(vr, acc_n.at[idx_d], add=True)
      pltpu.sync_copy(md, acc_d.at[idx_d8], add=True)
      for g in range(CHUNK // 16):
        rows = g * 16 + lax.iota(jnp.int32, 16)
        plsc.store_scatter(md, [rows, idx_d[pl.ds(g * 16, 16)] & 7],
                           jnp.zeros((16,), jnp.float32))

    plsc.subcore_barrier()
    pltpu.sync_copy(acc_n.at[pl.ds(s * r_tile, r_tile)],
                    onum_hbm.at[c].at[pl.ds(s * r_tile, r_tile)])
    pltpu.sync_copy(acc_d.at[pl.ds(s * rd_tile, rd_tile)],
                    oden_hbm.at[c].at[pl.ds(s * rd_tile, rd_tile)])

  num, den = k(src, dst, ktab, vtab, qtab, znum, zden)
  return num, den.reshape(NC, nden * DEN_W)


def _split_heads_w(w):
  return jnp.transpose(w.reshape(HID, HEADS, DH), (1, 0, 2))


def _fold_rel(w, b, rel, scale=None):
  wh = _split_heads_w(w)
  bh = b.reshape(HEADS, DH)
  wf = jnp.einsum("hde,hef->hdf", wh, rel)
  bf = jnp.einsum("he,hef->hf", bh, rel)
  if scale is not None:
    wf = wf * scale[:, None, None]
    bf = bf * scale[:, None]
  return wf, bf


def kernel(x_bus, x_gmd_bus, edge_index_bus_conn_bus,
           edge_index_bus_to_gmd_bus, edge_index_gmd_bus_from_bus, params):
  ei = {
      "bb": edge_index_bus_conn_bus,
      "bg": edge_index_bus_to_gmd_bus,
      "gb": edge_index_gmd_bus_from_bus,
  }
  ekey = {
      "bb": "bus__conn__bus",
      "bg": "bus__to__gmd_bus",
      "gb": "gmd_bus__from__bus",
  }
  h = {
      "bus": _embed(x_bus, params["lin"]["bus"]["W"],
                    params["lin"]["bus"]["b"]),
      "gmd": _embed(x_gmd_bus, params["lin"]["gmd_bus"]["W"],
                    params["lin"]["gmd_bus"]["b"]),
  }
  n_convs = len(params["convs"])
  inv_sqrt_dh = 1.0 / np.sqrt(DH)

  for li, cp in enumerate(params["convs"]):
    last = li == n_convs - 1
    scale = {e: cp["p_rel"][ekey[e]] * inv_sqrt_dh for e in ekey}

    def fold_kv(src_t, e):
      kw, kb = _fold_rel(cp["k"][src_t]["W"], cp["k"][src_t]["b"],
                         cp["a_rel"][ekey[e]], scale[e])
      vw, vb = _fold_rel(cp["v"][src_t]["W"], cp["v"][src_t]["b"],
                         cp["m_rel"][ekey[e]])
      return kw, kb, vw, vb

    qw_bus = _split_heads_w(cp["q"]["bus"]["W"])
    qb_bus = cp["q"]["bus"]["b"].reshape(HEADS, DH)
    kw_bb, kb_bb, vw_bb, vb_bb = fold_kv("bus", "bb")
    kw_gb, kb_gb, vw_gb, vb_gb = fold_kv("gmd_bus", "gb")
    if not last:
      kw_bg, kb_bg, vw_bg, vb_bg = fold_kv("bus", "bg")
      qw_gmd = _split_heads_w(cp["q"]["gmd_bus"]["W"])
      qb_gmd = cp["q"]["gmd_bus"]["b"].reshape(HEADS, DH)
      q_bus, k_bb, v_bb, k_bg, v_bg = _tables(
          h["bus"],
          [qw_bus, kw_bb, vw_bb, kw_bg, vw_bg],
          [qb_bus, kb_bb, vb_bb, kb_bg, vb_bg])
      q_gmd, k_gb, v_gb = _tables(
          h["gmd"], [qw_gmd, kw_gb, vw_gb], [qb_gmd, kb_gb, vb_gb])
    else:
      q_bus, k_bb, v_bb = _tables(
          h["bus"], [qw_bus, kw_bb, vw_bb], [qb_bus, kb_bb, vb_bb])
      k_gb, v_gb = _tables(h["gmd"], [kw_gb, vw_gb], [kb_gb, vb_gb])

    acc_bb = _edge_sc(ei["bb"][0], ei["bb"][1], k_bb, v_bb, q_bus,
                      N_BUS, N_BUS)
    acc_gb = _edge_sc(ei["gb"][0], ei["gb"][1], k_gb, v_gb, q_bus,
                      N_GMD, N_BUS)
    if not last:
      acc_bg = _edge_sc(ei["bg"][0], ei["bg"][1], k_bg, v_bg, q_gmd,
                        N_BUS, N_GMD)

    if not last:
      h = {
          "bus": _combine([acc_bb, acc_gb], h["bus"], cp["a"]["bus"]["W"],
                          cp["a"]["bus"]["b"], cp["skip"]["bus"]),
          "gmd": _combine([acc_bg], h["gmd"], cp["a"]["gmd_bus"]["W"],
                          cp["a"]["gmd_bus"]["b"], cp["skip"]["gmd_bus"]),
      }
    else:
      mlp = [(p["W"], p["b"]) for p in params["mlp"]]
      out = _combine([acc_bb, acc_gb], h["bus"], cp["a"]["bus"]["W"],
                     cp["a"]["bus"]["b"], cp["skip"]["bus"],
                     mlp_params=mlp)
  return out


# speedup vs baseline: 22.2486x; 1.2367x over previous
import functools

import jax
import jax.numpy as jnp
import numpy as np
from jax import lax
from jax.experimental import pallas as pl
from jax.experimental.pallas import tpu as pltpu
from jax.experimental.pallas import tpu_sc as plsc

N_BUS = 50000
N_GMD = 10000
D_IN = 128
HID = 64
HEADS = 2
DH = HID // HEADS
NC = 2
NS = 16
CHUNK = 128
DEN_W = 8
PREC = lax.Precision.HIGHEST
BN = 1000


def _relu_linear_body(x_ref, w_ref, b_ref, o_ref):
  o_ref[...] = jax.nn.relu(
      jnp.dot(x_ref[...], w_ref[...], precision=PREC) + b_ref[...])


def _embed(x, w, b):
  n, d = x.shape
  return pl.pallas_call(
      _relu_linear_body,
      grid=(n // BN,),
      in_specs=[
          pl.BlockSpec((BN, d), lambda i: (i, 0)),
          pl.BlockSpec((d, HID), lambda i: (0, 0)),
          pl.BlockSpec((1, HID), lambda i: (0, 0)),
      ],
      out_specs=pl.BlockSpec((BN, HID), lambda i: (i, 0)),
      out_shape=jax.ShapeDtypeStruct((n, HID), jnp.float32),
  )(x, w, b.reshape(1, HID))


def _tables_body(nroles, *refs):
  h_ref = refs[0]
  w_refs = refs[1:1 + nroles]
  b_refs = refs[1 + nroles:1 + 2 * nroles]
  o_refs = refs[1 + 2 * nroles:]
  hb = h_ref[...]
  for wr, br, orf in zip(w_refs, b_refs, o_refs):
    orf[...] = (jnp.dot(hb, wr[0], precision=PREC) + br[0])[None]


def _tables(h, ws, bs):
  n = h.shape[0]
  nroles = len(ws)
  widths = [w.shape[2] for w in ws]
  grid = (HEADS, n // BN)
  in_specs = [pl.BlockSpec((BN, HID), lambda hh, i: (i, 0))]
  in_specs += [pl.BlockSpec((1, HID, wd), lambda hh, i: (hh, 0, 0))
               for wd in widths]
  in_specs += [pl.BlockSpec((1, 1, wd), lambda hh, i: (hh, 0, 0))
               for wd in widths]
  out_specs = [pl.BlockSpec((1, BN, wd), lambda hh, i: (hh, i, 0))
               for wd in widths]
  out_shape = [jax.ShapeDtypeStruct((HEADS, n, wd), jnp.float32)
               for wd in widths]
  outs = pl.pallas_call(
      functools.partial(_tables_body, nroles),
      grid=grid,
      in_specs=in_specs,
      out_specs=out_specs,
      out_shape=out_shape,
  )(h, *ws, *[b.reshape(HEADS, 1, -1) for b in bs])
  return [o.reshape(HEADS * n, wd) for o, wd in zip(outs, widths)]


def _combine_body(ne, mlp, *refs):
  num_refs = refs[:ne]
  den_refs = refs[ne:2 * ne]
  ne2 = 2 * ne
  h_ref = refs[ne2]
  wa_ref, ba_ref, skip_ref = refs[ne2 + 1:ne2 + 4]
  o_ref = refs[-1]
  aggs = []
  for nr, dr in zip(num_refs, den_refs):
    num = nr[...]
    den = dr[...][:, :, None]
    agg = num / (den + 1e-16)
    aggs.append(jnp.concatenate([agg[0], agg[1]], axis=-1))
  m = aggs[0]
  for other in aggs[1:]:
    m = jnp.minimum(m, other)
  o = jnp.dot(jax.nn.gelu(m), wa_ref[...], precision=PREC) + ba_ref[...]
  gate = jax.nn.sigmoid(skip_ref[0, 0])
  h2 = gate * o + (1.0 - gate) * h_ref[...]
  if mlp:
    mrefs = refs[ne2 + 4:-1]
    for j in range(0, len(mrefs), 2):
      w, b = mrefs[j], mrefs[j + 1]
      h2 = jnp.dot(h2, w[...], precision=PREC) + b[...]
      if j + 2 < len(mrefs):
        h2 = jax.nn.relu(h2)
  o_ref[...] = h2


def _combine(accs, h_prev, wa, ba, skip, mlp_params=None):
  n = h_prev.shape[0]
  ne = len(accs)
  bc = 1024
  grid = (pl.cdiv(n, bc),)
  in_specs = [pl.BlockSpec((HEADS, bc, DH), lambda i: (0, i, 0))] * ne
  in_specs += [pl.BlockSpec((HEADS, bc), lambda i: (0, i))] * ne
  in_specs += [
      pl.BlockSpec((bc, HID), lambda i: (i, 0)),
      pl.BlockSpec((HID, HID), lambda i: (0, 0)),
      pl.BlockSpec((1, HID), lambda i: (0, 0)),
      pl.BlockSpec(memory_space=pltpu.SMEM),
  ]
  args = [*[a[0] for a in accs], *[a[1] for a in accs],
          h_prev, wa, ba.reshape(1, HID), skip.reshape(1, 1)]
  if mlp_params is not None:
    for w, b in mlp_params:
      dout = w.shape[1]
      in_specs += [
          pl.BlockSpec((HID, dout), lambda i: (0, 0)),
          pl.BlockSpec((1, dout), lambda i: (0, 0)),
      ]
      args += [w, b.reshape(1, dout)]
    dfin = mlp_params[-1][0].shape[1]
  else:
    dfin = HID
  return pl.pallas_call(
      functools.partial(_combine_body, ne, mlp_params is not None),
      grid=grid,
      in_specs=in_specs,
      out_specs=pl.BlockSpec((bc, dfin), lambda i: (i, 0)),
      out_shape=jax.ShapeDtypeStruct((n, dfin), jnp.float32),
  )(*args)


def _edge_sc(src, dst, kvtab, qtab, nsrc, ndst):
  chunk = 64 if ndst > 16384 else CHUNK
  e = src.shape[0]
  estep = NS * chunk * 2
  e_pad = ((e + estep - 1) // estep) * estep
  if e_pad != e:
    src = jnp.concatenate([src, jnp.zeros((e_pad - e,), jnp.int32)])
    dst = jnp.concatenate([dst, jnp.full((e_pad - e,), ndst, jnp.int32)])
  ndst_pad = ((ndst + 1 + 127) // 128) * 128
  nden = ndst_pad // DEN_W + 16
  nden = ((nden + 127) // 128) * 128
  r_tile = ndst_pad // NS
  rd_tile = nden // NS
  e_tile = e_pad // NS
  n_half = e_tile // (2 * chunk)
  znum = jnp.zeros((r_tile, DH), jnp.float32)
  zden = jnp.zeros((rd_tile, DEN_W), jnp.float32)
  mesh = plsc.VectorSubcoreMesh(
      core_axis_name="c", subcore_axis_name="s",
      num_cores=NC, num_subcores=NS)

  buf_scratch = [
      pltpu.VMEM((chunk,), jnp.int32),
      pltpu.VMEM((chunk,), jnp.int32),
      pltpu.VMEM((chunk,), jnp.int32),
      pltpu.VMEM((chunk,), jnp.int32),
      pltpu.VMEM((chunk, 2 * DH), jnp.float32),
      pltpu.VMEM((chunk, DH), jnp.float32),
      pltpu.VMEM((chunk, DH), jnp.float32),
      pltpu.VMEM((chunk, DEN_W), jnp.float32),
      pltpu.SemaphoreType.DMA,
      pltpu.SemaphoreType.DMA,
      pltpu.VMEM((chunk,), jnp.int32),
      pltpu.VMEM((chunk,), jnp.int32),
      pltpu.VMEM((chunk,), jnp.int32),
      pltpu.SemaphoreType.DMA,
  ]

  @functools.partial(
      pl.kernel,
      mesh=mesh,
      compiler_params=pltpu.CompilerParams(
          needs_layout_passes=False, use_tc_tiling_on_sc=False),
      out_type=(
          jax.ShapeDtypeStruct((NC, ndst_pad, DH), jnp.float32),
          jax.ShapeDtypeStruct((NC, nden, DEN_W), jnp.float32),
      ),
      scratch_types=[
          pltpu.VMEM_SHARED((ndst_pad, DH), jnp.float32),
          pltpu.VMEM_SHARED((nden, DEN_W), jnp.float32),
      ] + buf_scratch + buf_scratch,
  )
  def k(src_hbm, dst_hbm, kv_hbm, q_hbm, znum_hbm, zden_hbm,
        onum_hbm, oden_hbm, acc_n, acc_d, *bufs):
    c = lax.axis_index("c")
    s = lax.axis_index("s")
    B = [bufs[:14], bufs[14:]]
    zv = jnp.zeros((16,), jnp.float32)
    iota = lax.iota(jnp.int32, 16)
    base_t = s * e_tile

    pltpu.sync_copy(znum_hbm, acc_n.at[pl.ds(s * r_tile, r_tile)])
    pltpu.sync_copy(zden_hbm, acc_d.at[pl.ds(s * rd_tile, rd_tile)])
    for b in range(2):
      md = B[b][7]
      for g in range(chunk // 16):
        rows = g * 16 + iota
        for ch in range(DEN_W):
          plsc.store_scatter(md, [rows, jnp.full((16,), ch, jnp.int32)], zv)
    plsc.subcore_barrier()

    def fire_idx(j, buf):
      pltpu.async_copy(src_hbm.at[pl.ds(base_t + j * chunk, chunk)],
                       buf[10], buf[13])
      pltpu.async_copy(dst_hbm.at[pl.ds(base_t + j * chunk, chunk)],
                       buf[11], buf[13])

    def wait_idx(j, buf):
      pltpu.make_async_copy(src_hbm.at[pl.ds(base_t + j * chunk, chunk)],
                            buf[10], buf[13]).wait()
      pltpu.make_async_copy(dst_hbm.at[pl.ds(base_t + j * chunk, chunk)],
                            buf[11], buf[13]).wait()

    def prep_gather(buf):
      gsi, gdi, xsi, xdi, rdi = buf[0], buf[1], buf[10], buf[11], buf[12]
      for g in range(chunk // 16):
        dl = pl.ds(g * 16, 16)
        d = xdi[dl]
        gsi[dl] = xsi[dl] + c * nsrc
        gdi[dl] = jnp.minimum(d, ndst - 1) + c * ndst
        rdi[dl] = d

    def fire_gather(buf):
      pltpu.async_copy(kv_hbm.at[buf[0]], buf[4], buf[8])
      pltpu.async_copy(q_hbm.at[buf[1]], buf[5], buf[8])

    def wait_gather(buf):
      pltpu.make_async_copy(kv_hbm.at[buf[0]], buf[4], buf[8]).wait()
      pltpu.make_async_copy(q_hbm.at[buf[1]], buf[5], buf[8]).wait()

    def prep_scatter(buf):
      sdi, s8i, rdi = buf[2], buf[3], buf[12]
      for g in range(chunk // 16):
        dl = pl.ds(g * 16, 16)
        d = rdi[dl]
        sdi[dl] = d
        s8i[dl] = d >> 3

    def fire_scatter(buf):
      pltpu.async_copy(buf[6], acc_n.at[buf[2]], buf[9], add=True)
      pltpu.async_copy(buf[7], acc_d.at[buf[3]], buf[9], add=True)

    def wait_scatter(buf):
      pltpu.make_async_copy(buf[6], acc_n.at[buf[2]], buf[9]).wait()
      pltpu.make_async_copy(buf[7], acc_d.at[buf[3]], buf[9]).wait()

    def rezero_md(buf):
      sdi, md = buf[2], buf[7]
      for g in range(chunk // 16):
        rows = g * 16 + iota
        plsc.store_scatter(md, [rows, sdi[pl.ds(g * 16, 16)] & 7], zv)

    def compute(buf):
      kv, q, vr, md, sdi = buf[4], buf[5], buf[6], buf[7], buf[2]
      for g in range(chunk // 16):
        rows = g * 16 + iota
        a = jnp.zeros((16,), jnp.float32)
        for ch in range(DH):
          cc = jnp.full((16,), ch, jnp.int32)
          a = a + (plsc.load_gather(q, [rows, cc]) *
                   plsc.load_gather(kv, [rows, cc]))
        ev = jnp.exp(a)
        plsc.store_scatter(md, [rows, sdi[pl.ds(g * 16, 16)] & 7], ev)
        for ch in range(DH):
          cc = jnp.full((16,), ch, jnp.int32)
          cv = jnp.full((16,), DH + ch, jnp.int32)
          plsc.store_scatter(vr, [rows, cc],
                             plsc.load_gather(kv, [rows, cv]) * ev)

    for b in range(2):
      fire_idx(b, B[b])
      wait_idx(b, B[b])
      prep_gather(B[b])
      fire_gather(B[b])
      prep_scatter(B[b])
    if n_half > 1:
      fire_idx(2, B[0])
      fire_idx(3, B[1])

    @pl.loop(0, n_half)
    def _step(i):
      not_last = i < n_half - 1
      not_last2 = i < n_half - 2
      wait_gather(B[0])
      compute(B[0])
      fire_scatter(B[0])

      @pl.when(not_last)
      def _():
        wait_idx(2 * i + 2, B[0])
        prep_gather(B[0])
        fire_gather(B[0])

        @pl.when(not_last2)
        def _():
          fire_idx(2 * i + 4, B[0])

      @pl.when(i > 0)
      def _():
        wait_scatter(B[1])
        rezero_md(B[1])
        prep_scatter(B[1])

      wait_gather(B[1])
      compute(B[1])
      fire_scatter(B[1])

      @pl.when(not_last)
      def _():
        wait_idx(2 * i + 3, B[1])
        prep_gather(B[1])
        fire_gather(B[1])

        @pl.when(not_last2)
        def _():
          fire_idx(2 * i + 5, B[1])

      wait_scatter(B[0])

      @pl.when(not_last)
      def _():
        rezero_md(B[0])
        prep_scatter(B[0])

    wait_scatter(B[1])
    plsc.subcore_barrier()
    pltpu.sync_copy(acc_n.at[pl.ds(s * r_tile, r_tile)],
                    onum_hbm.at[c].at[pl.ds(s * r_tile, r_tile)])
    pltpu.sync_copy(acc_d.at[pl.ds(s * rd_tile, rd_tile)],
                    oden_hbm.at[c].at[pl.ds(s * rd_tile, rd_tile)])

  num, den = k(src, dst, kvtab, qtab, znum, zden)
  return num, den.reshape(NC, nden * DEN_W)


def _split_heads_w(w):
  return jnp.transpose(w.reshape(HID, HEADS, DH), (1, 0, 2))


def _fold_rel(w, b, rel, scale=None):
  wh = _split_heads_w(w)
  bh = b.reshape(HEADS, DH)
  wf = jnp.einsum("hde,hef->hdf", wh, rel)
  bf = jnp.einsum("he,hef->hf", bh, rel)
  if scale is not None:
    wf = wf * scale[:, None, None]
    bf = bf * scale[:, None]
  return wf, bf


def kernel(x_bus, x_gmd_bus, edge_index_bus_conn_bus,
           edge_index_bus_to_gmd_bus, edge_index_gmd_bus_from_bus, params):
  ei = {
      "bb": edge_index_bus_conn_bus,
      "bg": edge_index_bus_to_gmd_bus,
      "gb": edge_index_gmd_bus_from_bus,
  }
  ekey = {
      "bb": "bus__conn__bus",
      "bg": "bus__to__gmd_bus",
      "gb": "gmd_bus__from__bus",
  }
  h = {
      "bus": _embed(x_bus, params["lin"]["bus"]["W"],
                    params["lin"]["bus"]["b"]),
      "gmd": _embed(x_gmd_bus, params["lin"]["gmd_bus"]["W"],
                    params["lin"]["gmd_bus"]["b"]),
  }
  n_convs = len(params["convs"])
  inv_sqrt_dh = 1.0 / np.sqrt(DH)

  for li, cp in enumerate(params["convs"]):
    last = li == n_convs - 1
    scale = {e: cp["p_rel"][ekey[e]] * inv_sqrt_dh for e in ekey}

    def fold_kv(src_t, e):
      kw, kb = _fold_rel(cp["k"][src_t]["W"], cp["k"][src_t]["b"],
                         cp["a_rel"][ekey[e]], scale[e])
      vw, vb = _fold_rel(cp["v"][src_t]["W"], cp["v"][src_t]["b"],
                         cp["m_rel"][ekey[e]])
      return (jnp.concatenate([kw, vw], axis=2),
              jnp.concatenate([kb, vb], axis=1))

    qw_bus = _split_heads_w(cp["q"]["bus"]["W"])
    qb_bus = cp["q"]["bus"]["b"].reshape(HEADS, DH)
    kvw_bb, kvb_bb = fold_kv("bus", "bb")
    kvw_gb, kvb_gb = fold_kv("gmd_bus", "gb")
    if not last:
      kvw_bg, kvb_bg = fold_kv("bus", "bg")
      qw_gmd = _split_heads_w(cp["q"]["gmd_bus"]["W"])
      qb_gmd = cp["q"]["gmd_bus"]["b"].reshape(HEADS, DH)
      q_bus, kv_bb, kv_bg = _tables(
          h["bus"], [qw_bus, kvw_bb, kvw_bg], [qb_bus, kvb_bb, kvb_bg])
      q_gmd, kv_gb = _tables(
          h["gmd"], [qw_gmd, kvw_gb], [qb_gmd, kvb_gb])
    else:
      q_bus, kv_bb = _tables(h["bus"], [qw_bus, kvw_bb], [qb_bus, kvb_bb])
      (kv_gb,) = _tables(h["gmd"], [kvw_gb], [kvb_gb])

    acc_bb = _edge_sc(ei["bb"][0], ei["bb"][1], kv_bb, q_bus, N_BUS, N_BUS)
    acc_gb = _edge_sc(ei["gb"][0], ei["gb"][1], kv_gb, q_bus, N_GMD, N_BUS)
    if not last:
      acc_bg = _edge_sc(ei["bg"][0], ei["bg"][1], kv_bg, q_gmd,
                        N_BUS, N_GMD)

    if not last:
      h = {
          "bus": _combine([acc_bb, acc_gb], h["bus"], cp["a"]["bus"]["W"],
                          cp["a"]["bus"]["b"], cp["skip"]["bus"]),
          "gmd": _combine([acc_bg], h["gmd"], cp["a"]["gmd_bus"]["W"],
                          cp["a"]["gmd_bus"]["b"], cp["skip"]["gmd_bus"]),
      }
    else:
      mlp = [(p["W"], p["b"]) for p in params["mlp"]]
      out = _combine([acc_bb, acc_gb], h["bus"], cp["a"]["bus"]["W"],
                     cp["a"]["bus"]["b"], cp["skip"]["bus"],
                     mlp_params=mlp)
  return out


# speedup vs baseline: 22.6641x; 1.0187x over previous
import functools

import jax
import jax.numpy as jnp
import numpy as np
from jax import lax
from jax.experimental import pallas as pl
from jax.experimental.pallas import tpu as pltpu
from jax.experimental.pallas import tpu_sc as plsc

N_BUS = 50000
N_GMD = 10000
D_IN = 128
HID = 64
HEADS = 2
DH = HID // HEADS
NC = 2
NS = 16
CHUNK = 128
DEN_W = 8
PREC = lax.Precision.HIGHEST
BN = 1000


def _heads_out(hb, wr, br):
  return jnp.stack([
      jnp.dot(hb, wr[hh], precision=PREC) + br[hh]
      for hh in range(HEADS)])


def _embed_tables_body(nroles, x_ref, wl_ref, bl_ref, *refs):
  w_refs = refs[:nroles]
  b_refs = refs[nroles:2 * nroles]
  h_out = refs[2 * nroles]
  o_refs = refs[2 * nroles + 1:]
  hb = jax.nn.relu(
      jnp.dot(x_ref[...], wl_ref[...], precision=PREC) + bl_ref[...])
  h_out[...] = hb
  for wr, br, orf in zip(w_refs, b_refs, o_refs):
    orf[...] = _heads_out(hb, wr, br)


def _embed_tables(x, wl, bl, ws, bs):
  n, d = x.shape
  nroles = len(ws)
  widths = [w.shape[2] for w in ws]
  in_specs = [
      pl.BlockSpec((BN, d), lambda i: (i, 0)),
      pl.BlockSpec((d, HID), lambda i: (0, 0)),
      pl.BlockSpec((1, HID), lambda i: (0, 0)),
  ]
  in_specs += [pl.BlockSpec((HEADS, HID, wd), lambda i: (0, 0, 0))
               for wd in widths]
  in_specs += [pl.BlockSpec((HEADS, 1, wd), lambda i: (0, 0, 0))
               for wd in widths]
  out_specs = [pl.BlockSpec((BN, HID), lambda i: (i, 0))]
  out_specs += [pl.BlockSpec((HEADS, BN, wd), lambda i: (0, i, 0))
                for wd in widths]
  out_shape = [jax.ShapeDtypeStruct((n, HID), jnp.float32)]
  out_shape += [jax.ShapeDtypeStruct((HEADS, n, wd), jnp.float32)
                for wd in widths]
  outs = pl.pallas_call(
      functools.partial(_embed_tables_body, nroles),
      grid=(n // BN,),
      in_specs=in_specs,
      out_specs=out_specs,
      out_shape=out_shape,
  )(x, wl, bl.reshape(1, HID), *ws,
    *[b.reshape(HEADS, 1, -1) for b in bs])
  return outs[0], [o.reshape(HEADS * n, wd)
                   for o, wd in zip(outs[1:], widths)]


def _combine_body(ne, mlp, nroles, *refs):
  num_refs = refs[:ne]
  den_refs = refs[ne:2 * ne]
  ne2 = 2 * ne
  h_ref = refs[ne2]
  wa_ref, ba_ref, skip_ref = refs[ne2 + 1:ne2 + 4]
  rest = refs[ne2 + 4:]
  aggs = []
  for nr, dr in zip(num_refs, den_refs):
    num = nr[...]
    den = dr[...][:, :, None]
    agg = num / (den + 1e-16)
    aggs.append(jnp.concatenate([agg[0], agg[1]], axis=-1))
  m = aggs[0]
  for other in aggs[1:]:
    m = jnp.minimum(m, other)
  o = jnp.dot(jax.nn.gelu(m), wa_ref[...], precision=PREC) + ba_ref[...]
  gate = jax.nn.sigmoid(skip_ref[0, 0])
  h2 = gate * o + (1.0 - gate) * h_ref[...]
  if mlp:
    mrefs = rest[:-1]
    out = rest[-1]
    for j in range(0, len(mrefs), 2):
      w, b = mrefs[j], mrefs[j + 1]
      h2 = jnp.dot(h2, w[...], precision=PREC) + b[...]
      if j + 2 < len(mrefs):
        h2 = jax.nn.relu(h2)
    out[...] = h2
  else:
    w_refs = rest[:nroles]
    b_refs = rest[nroles:2 * nroles]
    h_out = rest[2 * nroles]
    o_refs = rest[2 * nroles + 1:]
    h_out[...] = h2
    for wr, br, orf in zip(w_refs, b_refs, o_refs):
      orf[...] = _heads_out(h2, wr, br)


def _combine(accs, h_prev, wa, ba, skip, mlp_params=None, ws=(), bs=()):
  n = h_prev.shape[0]
  ne = len(accs)
  nroles = len(ws)
  widths = [w.shape[2] for w in ws]
  bc = 1024
  grid = (pl.cdiv(n, bc),)
  in_specs = [pl.BlockSpec((HEADS, bc, DH), lambda i: (0, i, 0))] * ne
  in_specs += [pl.BlockSpec((HEADS, bc), lambda i: (0, i))] * ne
  in_specs += [
      pl.BlockSpec((bc, HID), lambda i: (i, 0)),
      pl.BlockSpec((HID, HID), lambda i: (0, 0)),
      pl.BlockSpec((1, HID), lambda i: (0, 0)),
      pl.BlockSpec(memory_space=pltpu.SMEM),
  ]
  args = [*[a[0] for a in accs], *[a[1] for a in accs],
          h_prev, wa, ba.reshape(1, HID), skip.reshape(1, 1)]
  if mlp_params is not None:
    for w, b in mlp_params:
      dout = w.shape[1]
      in_specs += [
          pl.BlockSpec((HID, dout), lambda i: (0, 0)),
          pl.BlockSpec((1, dout), lambda i: (0, 0)),
      ]
      args += [w, b.reshape(1, dout)]
    dfin = mlp_params[-1][0].shape[1]
    out_specs = pl.BlockSpec((bc, dfin), lambda i: (i, 0))
    out_shape = jax.ShapeDtypeStruct((n, dfin), jnp.float32)
    return pl.pallas_call(
        functools.partial(_combine_body, ne, True, 0),
        grid=grid,
        in_specs=in_specs,
        out_specs=out_specs,
        out_shape=out_shape,
    )(*args)
  in_specs += [pl.BlockSpec((HEADS, HID, wd), lambda i: (0, 0, 0))
               for wd in widths]
  in_specs += [pl.BlockSpec((HEADS, 1, wd), lambda i: (0, 0, 0))
               for wd in widths]
  args += list(ws) + [b.reshape(HEADS, 1, -1) for b in bs]
  out_specs = [pl.BlockSpec((bc, HID), lambda i: (i, 0))]
  out_specs += [pl.BlockSpec((HEADS, bc, wd), lambda i: (0, i, 0))
                for wd in widths]
  out_shape = [jax.ShapeDtypeStruct((n, HID), jnp.float32)]
  out_shape += [jax.ShapeDtypeStruct((HEADS, n, wd), jnp.float32)
                for wd in widths]
  outs = pl.pallas_call(
      functools.partial(_combine_body, ne, False, nroles),
      grid=grid,
      in_specs=in_specs,
      out_specs=out_specs,
      out_shape=out_shape,
  )(*args)
  return outs[0], [o.reshape(HEADS * n, wd)
                   for o, wd in zip(outs[1:], widths)]


def _edge_sc(src, dst, kvtab, qtab, nsrc, ndst):
  chunk = 64 if ndst > 16384 else CHUNK
  e = src.shape[0]
  estep = NS * chunk * 2
  e_pad = ((e + estep - 1) // estep) * estep
  if e_pad != e:
    src = jnp.concatenate([src, jnp.zeros((e_pad - e,), jnp.int32)])
    dst = jnp.concatenate([dst, jnp.full((e_pad - e,), ndst, jnp.int32)])
  ndst_pad = ((ndst + 1 + 127) // 128) * 128
  nden = ndst_pad // DEN_W + 16
  nden = ((nden + 127) // 128) * 128
  r_tile = ndst_pad // NS
  rd_tile = nden // NS
  e_tile = e_pad // NS
  n_half = e_tile // (2 * chunk)
  znum = jnp.zeros((r_tile, DH), jnp.float32)
  zden = jnp.zeros((rd_tile, DEN_W), jnp.float32)
  mesh = plsc.VectorSubcoreMesh(
      core_axis_name="c", subcore_axis_name="s",
      num_cores=NC, num_subcores=NS)

  buf_scratch = [
      pltpu.VMEM((chunk,), jnp.int32),
      pltpu.VMEM((chunk,), jnp.int32),
      pltpu.VMEM((chunk,), jnp.int32),
      pltpu.VMEM((chunk,), jnp.int32),
      pltpu.VMEM((chunk, 2 * DH), jnp.float32),
      pltpu.VMEM((chunk, DH), jnp.float32),
      pltpu.VMEM((chunk, DH), jnp.float32),
      pltpu.VMEM((chunk, DEN_W), jnp.float32),
      pltpu.SemaphoreType.DMA,
      pltpu.SemaphoreType.DMA,
      pltpu.VMEM((chunk,), jnp.int32),
      pltpu.VMEM((chunk,), jnp.int32),
      pltpu.VMEM((chunk,), jnp.int32),
      pltpu.SemaphoreType.DMA,
  ]

  @functools.partial(
      pl.kernel,
      mesh=mesh,
      compiler_params=pltpu.CompilerParams(
          needs_layout_passes=False, use_tc_tiling_on_sc=False),
      out_type=(
          jax.ShapeDtypeStruct((NC, ndst_pad, DH), jnp.float32),
          jax.ShapeDtypeStruct((NC, nden, DEN_W), jnp.float32),
      ),
      scratch_types=[
          pltpu.VMEM_SHARED((ndst_pad, DH), jnp.float32),
          pltpu.VMEM_SHARED((nden, DEN_W), jnp.float32),
      ] + buf_scratch + buf_scratch,
  )
  def k(src_hbm, dst_hbm, kv_hbm, q_hbm, znum_hbm, zden_hbm,
        onum_hbm, oden_hbm, acc_n, acc_d, *bufs):
    c = lax.axis_index("c")
    s = lax.axis_index("s")
    B = [bufs[:14], bufs[14:]]
    zv = jnp.zeros((16,), jnp.float32)
    iota = lax.iota(jnp.int32, 16)
    base_t = s * e_tile

    pltpu.sync_copy(znum_hbm, acc_n.at[pl.ds(s * r_tile, r_tile)])
    pltpu.sync_copy(zden_hbm, acc_d.at[pl.ds(s * rd_tile, rd_tile)])
    for b in range(2):
      md = B[b][7]
      for g in range(chunk // 16):
        rows = g * 16 + iota
        for ch in range(DEN_W):
          plsc.store_scatter(md, [rows, jnp.full((16,), ch, jnp.int32)], zv)
    plsc.subcore_barrier()

    def fire_idx(j, buf):
      pltpu.async_copy(src_hbm.at[pl.ds(base_t + j * chunk, chunk)],
                       buf[10], buf[13])
      pltpu.async_copy(dst_hbm.at[pl.ds(base_t + j * chunk, chunk)],
                       buf[11], buf[13])

    def wait_idx(j, buf):
      pltpu.make_async_copy(src_hbm.at[pl.ds(base_t + j * chunk, chunk)],
                            buf[10], buf[13]).wait()
      pltpu.make_async_copy(dst_hbm.at[pl.ds(base_t + j * chunk, chunk)],
                            buf[11], buf[13]).wait()

    def prep_gather(buf):
      gsi, gdi, xsi, xdi, rdi = buf[0], buf[1], buf[10], buf[11], buf[12]
      for g in range(chunk // 16):
        dl = pl.ds(g * 16, 16)
        d = xdi[dl]
        gsi[dl] = xsi[dl] + c * nsrc
        gdi[dl] = jnp.minimum(d, ndst - 1) + c * ndst
        rdi[dl] = d

    def fire_gather(buf):
      pltpu.async_copy(kv_hbm.at[buf[0]], buf[4], buf[8])
      pltpu.async_copy(q_hbm.at[buf[1]], buf[5], buf[8])

    def wait_gather(buf):
      pltpu.make_async_copy(kv_hbm.at[buf[0]], buf[4], buf[8]).wait()
      pltpu.make_async_copy(q_hbm.at[buf[1]], buf[5], buf[8]).wait()

    def prep_scatter(buf):
      sdi, s8i, rdi = buf[2], buf[3], buf[12]
      for g in range(chunk // 16):
        dl = pl.ds(g * 16, 16)
        d = rdi[dl]
        sdi[dl] = d
        s8i[dl] = d >> 3

    def fire_scatter(buf):
      pltpu.async_copy(buf[6], acc_n.at[buf[2]], buf[9], add=True)
      pltpu.async_copy(buf[7], acc_d.at[buf[3]], buf[9], add=True)

    def wait_scatter(buf):
      pltpu.make_async_copy(buf[6], acc_n.at[buf[2]], buf[9]).wait()
      pltpu.make_async_copy(buf[7], acc_d.at[buf[3]], buf[9]).wait()

    def rezero_md(buf):
      sdi, md = buf[2], buf[7]
      for g in range(chunk // 16):
        rows = g * 16 + iota
        plsc.store_scatter(md, [rows, sdi[pl.ds(g * 16, 16)] & 7], zv)

    def compute(buf):
      kv, q, vr, md, sdi = buf[4], buf[5], buf[6], buf[7], buf[2]
      for g in range(chunk // 16):
        rows = g * 16 + iota
        a = jnp.zeros((16,), jnp.float32)
        for ch in range(DH):
          cc = jnp.full((16,), ch, jnp.int32)
          a = a + (plsc.load_gather(q, [rows, cc]) *
                   plsc.load_gather(kv, [rows, cc]))
        ev = jnp.exp(a)
        plsc.store_scatter(md, [rows, sdi[pl.ds(g * 16, 16)] & 7], ev)
        for ch in range(DH):
          cc = jnp.full((16,), ch, jnp.int32)
          cv = jnp.full((16,), DH + ch, jnp.int32)
          plsc.store_scatter(vr, [rows, cc],
                             plsc.load_gather(kv, [rows, cv]) * ev)

    for b in range(2):
      fire_idx(b, B[b])
      wait_idx(b, B[b])
      prep_gather(B[b])
      fire_gather(B[b])
      prep_scatter(B[b])
    if n_half > 1:
      fire_idx(2, B[0])
      fire_idx(3, B[1])

    @pl.loop(0, n_half)
    def _step(i):
      not_last = i < n_half - 1
      not_last2 = i < n_half - 2
      wait_gather(B[0])
      compute(B[0])
      fire_scatter(B[0])

      @pl.when(not_last)
      def _():
        wait_idx(2 * i + 2, B[0])
        prep_gather(B[0])
        fire_gather(B[0])

        @pl.when(not_last2)
        def _():
          fire_idx(2 * i + 4, B[0])

      @pl.when(i > 0)
      def _():
        wait_scatter(B[1])
        rezero_md(B[1])
        prep_scatter(B[1])

      wait_gather(B[1])
      compute(B[1])
      fire_scatter(B[1])

      @pl.when(not_last)
      def _():
        wait_idx(2 * i + 3, B[1])
        prep_gather(B[1])
        fire_gather(B[1])

        @pl.when(not_last2)
        def _():
          fire_idx(2 * i + 5, B[1])

      wait_scatter(B[0])

      @pl.when(not_last)
      def _():
        rezero_md(B[0])
        prep_scatter(B[0])

    wait_scatter(B[1])
    plsc.subcore_barrier()
    pltpu.sync_copy(acc_n.at[pl.ds(s * r_tile, r_tile)],
                    onum_hbm.at[c].at[pl.ds(s * r_tile, r_tile)])
    pltpu.sync_copy(acc_d.at[pl.ds(s * rd_tile, rd_tile)],
                    oden_hbm.at[c].at[pl.ds(s * rd_tile, rd_tile)])

  num, den = k(src, dst, kvtab, qtab, znum, zden)
  return num, den.reshape(NC, nden * DEN_W)


def _split_heads_w(w):
  return jnp.transpose(w.reshape(HID, HEADS, DH), (1, 0, 2))


def _fold_rel(w, b, rel, scale=None):
  wh = _split_heads_w(w)
  bh = b.reshape(HEADS, DH)
  wf = jnp.einsum("hde,hef->hdf", wh, rel)
  bf = jnp.einsum("he,hef->hf", bh, rel)
  if scale is not None:
    wf = wf * scale[:, None, None]
    bf = bf * scale[:, None]
  return wf, bf


def kernel(x_bus, x_gmd_bus, edge_index_bus_conn_bus,
           edge_index_bus_to_gmd_bus, edge_index_gmd_bus_from_bus, params):
  ei = {
      "bb": edge_index_bus_conn_bus,
      "bg": edge_index_bus_to_gmd_bus,
      "gb": edge_index_gmd_bus_from_bus,
  }
  ekey = {
      "bb": "bus__conn__bus",
      "bg": "bus__to__gmd_bus",
      "gb": "gmd_bus__from__bus",
  }
  inv_sqrt_dh = 1.0 / np.sqrt(DH)

  def fold_kv(cp, src_t, e):
    scale = cp["p_rel"][ekey[e]] * inv_sqrt_dh
    kw, kb = _fold_rel(cp["k"][src_t]["W"], cp["k"][src_t]["b"],
                       cp["a_rel"][ekey[e]], scale)
    vw, vb = _fold_rel(cp["v"][src_t]["W"], cp["v"][src_t]["b"],
                       cp["m_rel"][ekey[e]])
    return (jnp.concatenate([kw, vw], axis=2),
            jnp.concatenate([kb, vb], axis=1))

  def fold_q(cp, t):
    return (_split_heads_w(cp["q"][t]["W"]),
            cp["q"][t]["b"].reshape(HEADS, DH))

  cp1, cp2 = params["convs"]
  qw1_bus, qb1_bus = fold_q(cp1, "bus")
  qw1_gmd, qb1_gmd = fold_q(cp1, "gmd_bus")
  kvw1_bb, kvb1_bb = fold_kv(cp1, "bus", "bb")
  kvw1_bg, kvb1_bg = fold_kv(cp1, "bus", "bg")
  kvw1_gb, kvb1_gb = fold_kv(cp1, "gmd_bus", "gb")
  qw2_bus, qb2_bus = fold_q(cp2, "bus")
  kvw2_bb, kvb2_bb = fold_kv(cp2, "bus", "bb")
  kvw2_gb, kvb2_gb = fold_kv(cp2, "gmd_bus", "gb")

  h_bus, (q1_bus, kv1_bb, kv1_bg) = _embed_tables(
      x_bus, params["lin"]["bus"]["W"], params["lin"]["bus"]["b"],
      [qw1_bus, kvw1_bb, kvw1_bg], [qb1_bus, kvb1_bb, kvb1_bg])
  h_gmd, (q1_gmd, kv1_gb) = _embed_tables(
      x_gmd_bus, params["lin"]["gmd_bus"]["W"], params["lin"]["gmd_bus"]["b"],
      [qw1_gmd, kvw1_gb], [qb1_gmd, kvb1_gb])

  acc_bb = _edge_sc(ei["bb"][0], ei["bb"][1], kv1_bb, q1_bus, N_BUS, N_BUS)
  acc_gb = _edge_sc(ei["gb"][0], ei["gb"][1], kv1_gb, q1_bus, N_GMD, N_BUS)
  acc_bg = _edge_sc(ei["bg"][0], ei["bg"][1], kv1_bg, q1_gmd, N_BUS, N_GMD)

  h1_bus, (q2_bus, kv2_bb) = _combine(
      [acc_bb, acc_gb], h_bus, cp1["a"]["bus"]["W"], cp1["a"]["bus"]["b"],
      cp1["skip"]["bus"], ws=[qw2_bus, kvw2_bb], bs=[qb2_bus, kvb2_bb])
  _, (kv2_gb,) = _combine(
      [acc_bg], h_gmd, cp1["a"]["gmd_bus"]["W"], cp1["a"]["gmd_bus"]["b"],
      cp1["skip"]["gmd_bus"], ws=[kvw2_gb], bs=[kvb2_gb])

  acc2_bb = _edge_sc(ei["bb"][0], ei["bb"][1], kv2_bb, q2_bus, N_BUS, N_BUS)
  acc2_gb = _edge_sc(ei["gb"][0], ei["gb"][1], kv2_gb, q2_bus, N_GMD, N_BUS)

  mlp = [(p["W"], p["b"]) for p in params["mlp"]]
  return _combine(
      [acc2_bb, acc2_gb], h1_bus, cp2["a"]["bus"]["W"], cp2["a"]["bus"]["b"],
      cp2["skip"]["bus"], mlp_params=mlp)


# speedup vs baseline: 22.7067x; 1.0019x over previous
import functools

import jax
import jax.numpy as jnp
import numpy as np
from jax import lax
from jax.experimental import pallas as pl
from jax.experimental.pallas import tpu as pltpu
from jax.experimental.pallas import tpu_sc as plsc

N_BUS = 50000
N_GMD = 10000
D_IN = 128
HID = 64
HEADS = 2
DH = HID // HEADS
NC = 2
NS = 16
CHUNK = 128
DEN_W = 8
PREC = lax.Precision.HIGHEST
BN = 1000


def _heads_out(hb, wr, br):
  return jnp.stack([
      jnp.dot(hb, wr[hh], precision=PREC) + br[hh]
      for hh in range(HEADS)])


def _embed_tables_body(nroles, x_ref, wl_ref, bl_ref, *refs):
  w_refs = refs[:nroles]
  b_refs = refs[nroles:2 * nroles]
  h_out = refs[2 * nroles]
  o_refs = refs[2 * nroles + 1:]
  hb = jax.nn.relu(
      jnp.dot(x_ref[...], wl_ref[...], precision=PREC) + bl_ref[...])
  h_out[...] = hb
  for wr, br, orf in zip(w_refs, b_refs, o_refs):
    orf[...] = _heads_out(hb, wr, br)


def _embed_tables(x, wl, bl, ws, bs):
  n, d = x.shape
  nroles = len(ws)
  widths = [w.shape[2] for w in ws]
  in_specs = [
      pl.BlockSpec((BN, d), lambda i: (i, 0)),
      pl.BlockSpec((d, HID), lambda i: (0, 0)),
      pl.BlockSpec((1, HID), lambda i: (0, 0)),
  ]
  in_specs += [pl.BlockSpec((HEADS, HID, wd), lambda i: (0, 0, 0))
               for wd in widths]
  in_specs += [pl.BlockSpec((HEADS, 1, wd), lambda i: (0, 0, 0))
               for wd in widths]
  out_specs = [pl.BlockSpec((BN, HID), lambda i: (i, 0))]
  out_specs += [pl.BlockSpec((HEADS, BN, wd), lambda i: (0, i, 0))
                for wd in widths]
  out_shape = [jax.ShapeDtypeStruct((n, HID), jnp.float32)]
  out_shape += [jax.ShapeDtypeStruct((HEADS, n, wd), jnp.float32)
                for wd in widths]
  outs = pl.pallas_call(
      functools.partial(_embed_tables_body, nroles),
      grid=(n // BN,),
      in_specs=in_specs,
      out_specs=out_specs,
      out_shape=out_shape,
  )(x, wl, bl.reshape(1, HID), *ws,
    *[b.reshape(HEADS, 1, -1) for b in bs])
  return outs[0], [o.reshape(HEADS * n, wd)
                   for o, wd in zip(outs[1:], widths)]


def _combine_body(ne, mlp, nroles, *refs):
  num_refs = refs[:ne]
  den_refs = refs[ne:2 * ne]
  ne2 = 2 * ne
  h_ref = refs[ne2]
  wa_ref, ba_ref, skip_ref = refs[ne2 + 1:ne2 + 4]
  rest = refs[ne2 + 4:]
  aggs = []
  for nr, dr in zip(num_refs, den_refs):
    num = nr[...]
    den = dr[...][:, :, None]
    agg = num / (den + 1e-16)
    aggs.append(jnp.concatenate([agg[0], agg[1]], axis=-1))
  m = aggs[0]
  for other in aggs[1:]:
    m = jnp.minimum(m, other)
  o = jnp.dot(jax.nn.gelu(m), wa_ref[...], precision=PREC) + ba_ref[...]
  gate = jax.nn.sigmoid(skip_ref[0, 0])
  h2 = gate * o + (1.0 - gate) * h_ref[...]
  if mlp:
    mrefs = rest[:-1]
    out = rest[-1]
    for j in range(0, len(mrefs), 2):
      w, b = mrefs[j], mrefs[j + 1]
      h2 = jnp.dot(h2, w[...], precision=PREC) + b[...]
      if j + 2 < len(mrefs):
        h2 = jax.nn.relu(h2)
    out[...] = h2
  else:
    w_refs = rest[:nroles]
    b_refs = rest[nroles:2 * nroles]
    h_out = rest[2 * nroles]
    o_refs = rest[2 * nroles + 1:]
    h_out[...] = h2
    for wr, br, orf in zip(w_refs, b_refs, o_refs):
      orf[...] = _heads_out(h2, wr, br)


def _combine(accs, h_prev, wa, ba, skip, mlp_params=None, ws=(), bs=()):
  n = h_prev.shape[0]
  ne = len(accs)
  nroles = len(ws)
  widths = [w.shape[2] for w in ws]
  bc = 1024
  grid = (pl.cdiv(n, bc),)
  in_specs = [pl.BlockSpec((HEADS, bc, DH), lambda i: (0, i, 0))] * ne
  in_specs += [pl.BlockSpec((HEADS, bc), lambda i: (0, i))] * ne
  in_specs += [
      pl.BlockSpec((bc, HID), lambda i: (i, 0)),
      pl.BlockSpec((HID, HID), lambda i: (0, 0)),
      pl.BlockSpec((1, HID), lambda i: (0, 0)),
      pl.BlockSpec(memory_space=pltpu.SMEM),
  ]
  args = [*[a[0] for a in accs], *[a[1] for a in accs],
          h_prev, wa, ba.reshape(1, HID), skip.reshape(1, 1)]
  if mlp_params is not None:
    for w, b in mlp_params:
      dout = w.shape[1]
      in_specs += [
          pl.BlockSpec((HID, dout), lambda i: (0, 0)),
          pl.BlockSpec((1, dout), lambda i: (0, 0)),
      ]
      args += [w, b.reshape(1, dout)]
    dfin = mlp_params[-1][0].shape[1]
    out_specs = pl.BlockSpec((bc, dfin), lambda i: (i, 0))
    out_shape = jax.ShapeDtypeStruct((n, dfin), jnp.float32)
    return pl.pallas_call(
        functools.partial(_combine_body, ne, True, 0),
        grid=grid,
        in_specs=in_specs,
        out_specs=out_specs,
        out_shape=out_shape,
    )(*args)
  in_specs += [pl.BlockSpec((HEADS, HID, wd), lambda i: (0, 0, 0))
               for wd in widths]
  in_specs += [pl.BlockSpec((HEADS, 1, wd), lambda i: (0, 0, 0))
               for wd in widths]
  args += list(ws) + [b.reshape(HEADS, 1, -1) for b in bs]
  out_specs = [pl.BlockSpec((bc, HID), lambda i: (i, 0))]
  out_specs += [pl.BlockSpec((HEADS, bc, wd), lambda i: (0, i, 0))
                for wd in widths]
  out_shape = [jax.ShapeDtypeStruct((n, HID), jnp.float32)]
  out_shape += [jax.ShapeDtypeStruct((HEADS, n, wd), jnp.float32)
                for wd in widths]
  outs = pl.pallas_call(
      functools.partial(_combine_body, ne, False, nroles),
      grid=grid,
      in_specs=in_specs,
      out_specs=out_specs,
      out_shape=out_shape,
  )(*args)
  return outs[0], [o.reshape(HEADS * n, wd)
                   for o, wd in zip(outs[1:], widths)]


def _edge_sc(src, dst, kvtab, qtab, nsrc, ndst):
  chunk = 64 if ndst > 16384 else CHUNK
  e = src.shape[0]
  estep = NS * chunk * 2
  e_pad = ((e + estep - 1) // estep) * estep
  if e_pad != e:
    src = jnp.concatenate([src, jnp.zeros((e_pad - e,), jnp.int32)])
    dst = jnp.concatenate([dst, jnp.full((e_pad - e,), ndst, jnp.int32)])
  ndst_pad = ((ndst + 1 + 127) // 128) * 128
  nden = ndst_pad // DEN_W + 16
  nden = ((nden + 127) // 128) * 128
  r_tile = ndst_pad // NS
  rd_tile = nden // NS
  e_tile = e_pad // NS
  n_half = e_tile // (2 * chunk)
  znum = jnp.zeros((r_tile, DH), jnp.float32)
  zden = jnp.zeros((rd_tile, DEN_W), jnp.float32)
  mesh = plsc.VectorSubcoreMesh(
      core_axis_name="c", subcore_axis_name="s",
      num_cores=NC, num_subcores=NS)

  buf_scratch = [
      pltpu.VMEM((chunk,), jnp.int32),
      pltpu.VMEM((chunk,), jnp.int32),
      pltpu.VMEM((chunk,), jnp.int32),
      pltpu.VMEM((chunk,), jnp.int32),
      pltpu.VMEM((chunk, 2 * DH), jnp.float32),
      pltpu.VMEM((chunk, DH), jnp.float32),
      pltpu.VMEM((chunk, DH), jnp.float32),
      pltpu.VMEM((chunk, DEN_W), jnp.float32),
      pltpu.SemaphoreType.DMA,
      pltpu.SemaphoreType.DMA,
      pltpu.VMEM((chunk,), jnp.int32),
      pltpu.VMEM((chunk,), jnp.int32),
      pltpu.VMEM((chunk,), jnp.int32),
      pltpu.SemaphoreType.DMA,
  ]

  @functools.partial(
      pl.kernel,
      mesh=mesh,
      compiler_params=pltpu.CompilerParams(
          needs_layout_passes=False, use_tc_tiling_on_sc=False),
      out_type=(
          jax.ShapeDtypeStruct((NC, ndst_pad, DH), jnp.float32),
          jax.ShapeDtypeStruct((NC, nden, DEN_W), jnp.float32),
      ),
      scratch_types=[
          pltpu.VMEM_SHARED((ndst_pad, DH), jnp.float32),
          pltpu.VMEM_SHARED((nden, DEN_W), jnp.float32),
      ] + buf_scratch + buf_scratch,
  )
  def k(src_hbm, dst_hbm, kv_hbm, q_hbm, znum_hbm, zden_hbm,
        onum_hbm, oden_hbm, acc_n, acc_d, *bufs):
    c = lax.axis_index("c")
    s = lax.axis_index("s")
    B = [bufs[:14], bufs[14:]]
    zv = jnp.zeros((16,), jnp.float32)
    iota = lax.iota(jnp.int32, 16)
    base_t = s * e_tile

    pltpu.sync_copy(znum_hbm, acc_n.at[pl.ds(s * r_tile, r_tile)])
    pltpu.sync_copy(zden_hbm, acc_d.at[pl.ds(s * rd_tile, rd_tile)])
    for b in range(2):
      md = B[b][7]
      for g in range(chunk // 16):
        rows = g * 16 + iota
        for ch in range(DEN_W):
          plsc.store_scatter(md, [rows, jnp.full((16,), ch, jnp.int32)], zv)
    plsc.subcore_barrier()

    def fire_idx(j, buf):
      pltpu.async_copy(src_hbm.at[pl.ds(base_t + j * chunk, chunk)],
                       buf[10], buf[13])
      pltpu.async_copy(dst_hbm.at[pl.ds(base_t + j * chunk, chunk)],
                       buf[11], buf[13])

    def wait_idx(j, buf):
      pltpu.make_async_copy(src_hbm.at[pl.ds(base_t + j * chunk, chunk)],
                            buf[10], buf[13]).wait()
      pltpu.make_async_copy(dst_hbm.at[pl.ds(base_t + j * chunk, chunk)],
                            buf[11], buf[13]).wait()

    def prep_gather(buf):
      gsi, gdi, xsi, xdi, rdi = buf[0], buf[1], buf[10], buf[11], buf[12]
      for g in range(chunk // 16):
        dl = pl.ds(g * 16, 16)
        d = xdi[dl]
        gsi[dl] = xsi[dl] + c * nsrc
        gdi[dl] = jnp.minimum(d, ndst - 1) + c * ndst
        rdi[dl] = d

    def fire_gather(buf):
      pltpu.async_copy(kv_hbm.at[buf[0]], buf[4], buf[8])
      pltpu.async_copy(q_hbm.at[buf[1]], buf[5], buf[8])

    def wait_gather(buf):
      pltpu.make_async_copy(kv_hbm.at[buf[0]], buf[4], buf[8]).wait()
      pltpu.make_async_copy(q_hbm.at[buf[1]], buf[5], buf[8]).wait()

    def prep_scatter(buf):
      sdi, s8i, rdi = buf[2], buf[3], buf[12]
      for g in range(chunk // 16):
        dl = pl.ds(g * 16, 16)
        d = rdi[dl]
        sdi[dl] = d
        s8i[dl] = d >> 3

    def fire_scatter(buf):
      pltpu.async_copy(buf[6], acc_n.at[buf[2]], buf[9], add=True)
      pltpu.async_copy(buf[7], acc_d.at[buf[3]], buf[9], add=True)

    def wait_scatter(buf):
      pltpu.make_async_copy(buf[6], acc_n.at[buf[2]], buf[9]).wait()
      pltpu.make_async_copy(buf[7], acc_d.at[buf[3]], buf[9]).wait()

    def rezero_md(buf):
      sdi, md = buf[2], buf[7]
      for g in range(chunk // 16):
        rows = g * 16 + iota
        plsc.store_scatter(md, [rows, sdi[pl.ds(g * 16, 16)] & 7], zv)

    def compute(buf):
      kv, q, vr, md, sdi = buf[4], buf[5], buf[6], buf[7], buf[2]
      for g in range(chunk // 16):
        rows = g * 16 + iota
        a = jnp.zeros((16,), jnp.float32)
        for ch in range(DH):
          cc = jnp.full((16,), ch, jnp.int32)
          a = a + (plsc.load_gather(q, [rows, cc]) *
                   plsc.load_gather(kv, [rows, cc]))
        ev = jnp.exp(a)
        plsc.store_scatter(md, [rows, sdi[pl.ds(g * 16, 16)] & 7], ev)
        for ch in range(DH):
          cc = jnp.full((16,), ch, jnp.int32)
          cv = jnp.full((16,), DH + ch, jnp.int32)
          plsc.store_scatter(vr, [rows, cc],
                             plsc.load_gather(kv, [rows, cv]) * ev)

    for b in range(2):
      fire_idx(b, B[b])
      wait_idx(b, B[b])
      prep_gather(B[b])
      fire_gather(B[b])
      prep_scatter(B[b])
    if n_half > 1:
      fire_idx(2, B[0])
      fire_idx(3, B[1])

    @pl.loop(0, n_half)
    def _step(i):
      not_last = i < n_half - 1
      not_last2 = i < n_half - 2
      wait_gather(B[0])
      compute(B[0])
      fire_scatter(B[0])

      @pl.when(not_last)
      def _():
        wait_idx(2 * i + 2, B[0])
        prep_gather(B[0])
        fire_gather(B[0])

        @pl.when(not_last2)
        def _():
          fire_idx(2 * i + 4, B[0])

      @pl.when(i > 0)
      def _():
        wait_scatter(B[1])
        rezero_md(B[1])
        prep_scatter(B[1])

      wait_gather(B[1])
      compute(B[1])
      fire_scatter(B[1])

      @pl.when(not_last)
      def _():
        wait_idx(2 * i + 3, B[1])
        prep_gather(B[1])
        fire_gather(B[1])

        @pl.when(not_last2)
        def _():
          fire_idx(2 * i + 5, B[1])

      wait_scatter(B[0])

      @pl.when(not_last)
      def _():
        rezero_md(B[0])
        prep_scatter(B[0])

      plsc.subcore_barrier()

    wait_scatter(B[1])
    plsc.subcore_barrier()
    pltpu.sync_copy(acc_n.at[pl.ds(s * r_tile, r_tile)],
                    onum_hbm.at[c].at[pl.ds(s * r_tile, r_tile)])
    pltpu.sync_copy(acc_d.at[pl.ds(s * rd_tile, rd_tile)],
                    oden_hbm.at[c].at[pl.ds(s * rd_tile, rd_tile)])

  num, den = k(src, dst, kvtab, qtab, znum, zden)
  return num, den.reshape(NC, nden * DEN_W)


def _split_heads_w(w):
  return jnp.transpose(w.reshape(HID, HEADS, DH), (1, 0, 2))


def _fold_rel(w, b, rel, scale=None):
  wh = _split_heads_w(w)
  bh = b.reshape(HEADS, DH)
  wf = jnp.einsum("hde,hef->hdf", wh, rel)
  bf = jnp.einsum("he,hef->hf", bh, rel)
  if scale is not None:
    wf = wf * scale[:, None, None]
    bf = bf * scale[:, None]
  return wf, bf


def kernel(x_bus, x_gmd_bus, edge_index_bus_conn_bus,
           edge_index_bus_to_gmd_bus, edge_index_gmd_bus_from_bus, params):
  ei = {
      "bb": edge_index_bus_conn_bus,
      "bg": edge_index_bus_to_gmd_bus,
      "gb": edge_index_gmd_bus_from_bus,
  }
  ekey = {
      "bb": "bus__conn__bus",
      "bg": "bus__to__gmd_bus",
      "gb": "gmd_bus__from__bus",
  }
  inv_sqrt_dh = 1.0 / np.sqrt(DH)

  def fold_kv(cp, src_t, e):
    scale = cp["p_rel"][ekey[e]] * inv_sqrt_dh
    kw, kb = _fold_rel(cp["k"][src_t]["W"], cp["k"][src_t]["b"],
                       cp["a_rel"][ekey[e]], scale)
    vw, vb = _fold_rel(cp["v"][src_t]["W"], cp["v"][src_t]["b"],
                       cp["m_rel"][ekey[e]])
    return (jnp.concatenate([kw, vw], axis=2),
            jnp.concatenate([kb, vb], axis=1))

  def fold_q(cp, t):
    return (_split_heads_w(cp["q"][t]["W"]),
            cp["q"][t]["b"].reshape(HEADS, DH))

  cp1, cp2 = params["convs"]
  qw1_bus, qb1_bus = fold_q(cp1, "bus")
  qw1_gmd, qb1_gmd = fold_q(cp1, "gmd_bus")
  kvw1_bb, kvb1_bb = fold_kv(cp1, "bus", "bb")
  kvw1_bg, kvb1_bg = fold_kv(cp1, "bus", "bg")
  kvw1_gb, kvb1_gb = fold_kv(cp1, "gmd_bus", "gb")
  qw2_bus, qb2_bus = fold_q(cp2, "bus")
  kvw2_bb, kvb2_bb = fold_kv(cp2, "bus", "bb")
  kvw2_gb, kvb2_gb = fold_kv(cp2, "gmd_bus", "gb")

  h_bus, (q1_bus, kv1_bb, kv1_bg) = _embed_tables(
      x_bus, params["lin"]["bus"]["W"], params["lin"]["bus"]["b"],
      [qw1_bus, kvw1_bb, kvw1_bg], [qb1_bus, kvb1_bb, kvb1_bg])
  h_gmd, (q1_gmd, kv1_gb) = _embed_tables(
      x_gmd_bus, params["lin"]["gmd_bus"]["W"], params["lin"]["gmd_bus"]["b"],
      [qw1_gmd, kvw1_gb], [qb1_gmd, kvb1_gb])

  acc_bb = _edge_sc(ei["bb"][0], ei["bb"][1], kv1_bb, q1_bus, N_BUS, N_BUS)
  acc_gb = _edge_sc(ei["gb"][0], ei["gb"][1], kv1_gb, q1_bus, N_GMD, N_BUS)
  acc_bg = _edge_sc(ei["bg"][0], ei["bg"][1], kv1_bg, q1_gmd, N_BUS, N_GMD)

  h1_bus, (q2_bus, kv2_bb) = _combine(
      [acc_bb, acc_gb], h_bus, cp1["a"]["bus"]["W"], cp1["a"]["bus"]["b"],
      cp1["skip"]["bus"], ws=[qw2_bus, kvw2_bb], bs=[qb2_bus, kvb2_bb])
  _, (kv2_gb,) = _combine(
      [acc_bg], h_gmd, cp1["a"]["gmd_bus"]["W"], cp1["a"]["gmd_bus"]["b"],
      cp1["skip"]["gmd_bus"], ws=[kvw2_gb], bs=[kvb2_gb])

  acc2_bb = _edge_sc(ei["bb"][0], ei["bb"][1], kv2_bb, q2_bus, N_BUS, N_BUS)
  acc2_gb = _edge_sc(ei["gb"][0], ei["gb"][1], kv2_gb, q2_bus, N_GMD, N_BUS)

  mlp = [(p["W"], p["b"]) for p in params["mlp"]]
  return _combine(
      [acc2_bb, acc2_gb], h1_bus, cp2["a"]["bus"]["W"], cp2["a"]["bus"]["b"],
      cp2["skip"]["bus"], mlp_params=mlp)
